# Initial kernel scaffold; baseline (speedup 1.0000x reference)
import jax
import jax.numpy as jnp
from jax.experimental import pallas as pl


def kernel(x, mark, edge_index, edge_attr, batch, scen_adj, W_var, b_var, W_con, b_con, W1, b1, W2, b2, W3, b3, g1, be1, g2, be2, g3, be3):
    raise NotImplementedError("write your pallas kernel here")


# speedup vs baseline: 2.5485x; 2.5485x over previous
import functools

import jax
import jax.numpy as jnp
from jax import lax
from jax.experimental import pallas as pl
from jax.experimental.pallas import tpu as pltpu

_INTERPRET = False

N = 10000
NP = 10240
D = 128
H = 128
S = 512
SP = 640


def _k1_body(x_ref, markc_ref, mark2_ref, wv_ref, bv_ref, wc_ref, bc_ref,
             g1_ref, be1_ref, w1_ref, z_ref, inv_ref):
    x = x_ref[...]
    markc = markc_ref[...]
    var_all = jnp.tanh(jnp.dot(x, wv_ref[...]) + bv_ref[...])
    con_all = jnp.tanh(x[:, 0:1] * wc_ref[...] + bc_ref[...])
    feats = jnp.where(markc == 1, con_all, var_all)
    rows = lax.broadcasted_iota(jnp.int32, (NP, 1), 0)
    real = rows < N
    fm = jnp.where(real, feats, 0.0)
    m = jnp.sum(fm, axis=0, keepdims=True) / N
    v = jnp.sum(jnp.where(real, (feats - m) ** 2, 0.0), axis=0,
                keepdims=True) / N
    fbn = (feats - m) * lax.rsqrt(v + 1e-5) * g1_ref[...] + be1_ref[...]
    z_ref[...] = jnp.dot(fbn, w1_ref[...])

    mc = (mark2_ref[...] == 1).astype(jnp.float32)
    ii = lax.broadcasted_iota(jnp.int32, (128, 128), 0)
    jj = lax.broadcasted_iota(jnp.int32, (128, 128), 1)
    u_tri = (ii <= jj).astype(jnp.float32)
    rowcum = jnp.dot(mc, u_tri)
    s = rowcum[:, 127:128]
    ri = lax.broadcasted_iota(jnp.int32, (80, 80), 0)
    rj = lax.broadcasted_iota(jnp.int32, (80, 80), 1)
    l_tri = (rj < ri).astype(jnp.float32)
    p = jnp.dot(l_tri, s)
    cc = rowcum + p
    ncon = p[79:80, 0:1] + s[79:80, 0:1]
    gi = (lax.broadcasted_iota(jnp.int32, (80, 128), 0) * 128
          + lax.broadcasted_iota(jnp.int32, (80, 128), 1)).astype(jnp.float32)
    inv = jnp.where(mc == 1.0, cc - 1.0, ncon + gi - cc)
    inv_ref[...] = inv.astype(jnp.int32)


def _k1(x_pad, mark_pad, W_var, b_var, W_con, b_con, g1, be1, W1):
    return pl.pallas_call(
        _k1_body,
        out_shape=(jax.ShapeDtypeStruct((NP, H), jnp.float32),
                   jax.ShapeDtypeStruct((80, 128), jnp.int32)),
        interpret=_INTERPRET,
    )(x_pad, mark_pad.reshape(NP, 1), mark_pad.reshape(80, 128),
      W_var, b_var.reshape(1, H), W_con, b_con.reshape(1, H),
      g1.reshape(1, H), be1.reshape(1, H), W1)


def _k3_body(acc_ref, dis_ref, b1_ref, g2_ref, be2_ref, h2_ref):
    y = dis_ref[...] * acc_ref[...] + b1_ref[...]
    rows = lax.broadcasted_iota(jnp.int32, (NP, 1), 0)
    real = rows < N
    ym = jnp.where(real, y, 0.0)
    m = jnp.sum(ym, axis=0, keepdims=True) / N
    v = jnp.sum(jnp.where(real, (y - m) ** 2, 0.0), axis=0, keepdims=True) / N
    h2_ref[...] = jnp.tanh((y - m) * lax.rsqrt(v + 1e-5) * g2_ref[...]
                           + be2_ref[...])


def _k3(acc, dis, b1, g2, be2):
    return pl.pallas_call(
        _k3_body,
        out_shape=jax.ShapeDtypeStruct((NP, H), jnp.float32),
        interpret=_INTERPRET,
    )(acc, dis.reshape(NP, 1), b1.reshape(1, H), g2.reshape(1, H),
      be2.reshape(1, H))


def _k4_body(psum_ref, cnt_ref, adj_ref, w2_ref, b2_ref, w3_ref, b3_ref,
             g3_ref, be3_ref, feat_ref, mean_ref):
    pooled = psum_ref[...] / jnp.maximum(cnt_ref[...], 1.0)
    a = (adj_ref[...] >= 0.7).astype(jnp.float32)
    ri = lax.broadcasted_iota(jnp.int32, (S, S), 0)
    ci = lax.broadcasted_iota(jnp.int32, (S, S), 1)
    ah = a + (ri == ci).astype(jnp.float32)
    deg_row = jnp.sum(ah, axis=0, keepdims=True)
    ones_col = jnp.ones((S, 1), jnp.float32)
    deg_col = lax.dot_general(ah, ones_col,
                              (((0,), (0,)), ((), ())))
    m = lax.rsqrt(deg_col) * ah * lax.rsqrt(deg_row)
    xw2 = jnp.dot(pooled, w2_ref[...])
    out2 = lax.dot_general(m, xw2, (((0,), (0,)), ((), ())), precision=lax.Precision.HIGHEST) + b2_ref[...]
    mm = jnp.mean(out2, axis=0, keepdims=True)
    vv = jnp.mean((out2 - mm) ** 2, axis=0, keepdims=True)
    f1 = jnp.tanh((out2 - mm) * lax.rsqrt(vv + 1e-5) * g3_ref[...]
                  + be3_ref[...])
    xw3 = jnp.dot(f1, w3_ref[...])
    out3 = jnp.tanh(lax.dot_general(m, xw3, (((0,), (0,)), ((), ())), precision=lax.Precision.HIGHEST)
                    + b3_ref[...])
    feat_ref[...] = out3
    mean_ref[...] = jnp.mean(out3, axis=0, keepdims=True)


def _k4(pool_sum, cnt, scen_adj, W2, b2, W3, b3, g3, be3):
    return pl.pallas_call(
        _k4_body,
        out_shape=(jax.ShapeDtypeStruct((S, H), jnp.float32),
                   jax.ShapeDtypeStruct((1, H), jnp.float32)),
        interpret=_INTERPRET,
    )(pool_sum, cnt, scen_adj, W2, b2.reshape(1, H), W3, b3.reshape(1, H),
      g3.reshape(1, H), be3.reshape(1, H))


def kernel(x, mark, edge_index, edge_attr, batch, scen_adj,
           W_var, b_var, W_con, b_con, W1, b1, W2, b2, W3, b3,
           g1, be1, g2, be2, g3, be3):
    f32, i32 = jnp.float32, jnp.int32
    mark = mark.astype(i32)

    x_pad = jnp.concatenate([x, jnp.zeros((NP - N, D), f32)])
    mark_pad = jnp.concatenate([mark, jnp.full((NP - N,), 2, i32)])

    EP = 331776
    npad_e = EP - 320000 - N
    dummy = (N + (jnp.arange(npad_e, dtype=i32) % (NP - N)))
    row_p = jnp.concatenate([edge_index[0].astype(i32),
                             jnp.arange(N, dtype=i32), dummy])
    col_p = jnp.concatenate([edge_index[1].astype(i32),
                             jnp.arange(N, dtype=i32), dummy])
    ew_p = jnp.concatenate([edge_attr.astype(f32), jnp.ones((N,), f32),
                            jnp.zeros((npad_e,), f32)])
    batch_pad = jnp.concatenate(
        [batch.astype(i32), S + (jnp.arange(NP - N, dtype=i32) % (SP - S))])

    z, inv2 = _k1(x_pad, mark_pad, W_var, b_var, W_con, b_con, g1, be1, W1)
    inv = inv2.reshape(NP)

    deg = jax.ops.segment_sum(ew_p, col_p, num_segments=NP)
    dis = jnp.where(deg > 0, lax.rsqrt(jnp.maximum(deg, 1e-30)), 0.0)
    z_p = jnp.zeros((NP, H), f32).at[inv].set(z)
    w_e = ew_p * dis[row_p]
    acc = jax.ops.segment_sum(w_e[:, None] * z_p[row_p], col_p,
                              num_segments=NP)

    h2 = _k3(acc, dis, b1, g2, be2)

    pool_sum = jax.ops.segment_sum(h2, batch_pad, num_segments=SP)[:S]
    cnt = jax.ops.segment_sum(jnp.ones((NP,), f32), batch_pad,
                              num_segments=SP)[:S]

    feat, mean = _k4(pool_sum, cnt.reshape(S, 1), scen_adj, W2, b2, W3, b3,
                     g3, be3)
    return (feat, mean.reshape(H))


# speedup vs baseline: 21.2373x; 8.3331x over previous
import functools

import jax
import jax.numpy as jnp
from jax import lax
from jax.experimental import pallas as pl
from jax.experimental.pallas import tpu as pltpu
from jax.experimental.pallas import tpu_sc as plsc

_INTERPRET = False

N = 10000
NP = 10240
D = 128
H = 128
S = 512
SP = 640
EP = 331776
EPW = EP // 32
NW_EDGE = EPW // 128
ROWS_W = NP // 32

_SC_MESH = plsc.VectorSubcoreMesh(core_axis_name="c", subcore_axis_name="s")


def _sc1_body(colr, ewr, zr, invr, zerosr, degout, zpout,
              degacc, colbuf, ewbuf, zbuf, invbuf, sem):
    c = lax.axis_index("c")
    s = lax.axis_index("s")
    wid = c * 16 + s

    @pl.when(s == 0)
    def _():
        pltpu.sync_copy(zerosr, degacc)

    plsc.subcore_barrier()
    ebase = wid * EPW

    @pl.loop(0, NW_EDGE)
    def _(j):
        off = ebase + j * 128
        pltpu.sync_copy(colr.at[pl.ds(off, 128)], colbuf)
        pltpu.sync_copy(ewr.at[pl.ds(off, 128)], ewbuf)
        pltpu.sync_copy(ewbuf, degacc.at[colbuf], add=True)

    rbase = wid * ROWS_W
    for q in range(4):
        pltpu.sync_copy(invr.at[pl.ds(rbase + 80 * q, 80)], invbuf)
        pltpu.sync_copy(zr.at[pl.ds(rbase + 80 * q, 80)], zbuf)
        pltpu.async_copy(zbuf, zpout.at[invbuf], sem).wait()

    plsc.subcore_barrier()

    @pl.when(s == 0)
    def _():
        pltpu.sync_copy(degacc, degout.at[c])


def _sc1(col_p, ew_p, z, inv, zeros_np):
    return pl.kernel(
        _sc1_body,
        out_type=(jax.ShapeDtypeStruct((2, NP), jnp.float32),
                  jax.ShapeDtypeStruct((NP, H), jnp.float32)),
        mesh=_SC_MESH,
        scratch_types=[
            pltpu.VMEM_SHARED((NP,), jnp.float32),
            pltpu.VMEM((128,), jnp.int32),
            pltpu.VMEM((128,), jnp.float32),
            pltpu.VMEM((80, H), jnp.float32),
            pltpu.VMEM((80,), jnp.int32),
            pltpu.SemaphoreType.DMA,
        ],
    )(col_p, ew_p, z, inv, zeros_np)


def _k1_body(x_ref, markc_ref, mark2_ref, wv_ref, bv_ref, wc_ref, bc_ref,
             g1_ref, be1_ref, w1_ref, z_ref, inv_ref):
    x = x_ref[...]
    markc = markc_ref[...]
    var_all = jnp.tanh(jnp.dot(x, wv_ref[...]) + bv_ref[...])
    con_all = jnp.tanh(x[:, 0:1] * wc_ref[...] + bc_ref[...])
    feats = jnp.where(markc == 1, con_all, var_all)
    rows = lax.broadcasted_iota(jnp.int32, (NP, 1), 0)
    real = rows < N
    fm = jnp.where(real, feats, 0.0)
    m = jnp.sum(fm, axis=0, keepdims=True) / N
    v = jnp.sum(jnp.where(real, (feats - m) ** 2, 0.0), axis=0,
                keepdims=True) / N
    fbn = (feats - m) * lax.rsqrt(v + 1e-5) * g1_ref[...] + be1_ref[...]
    z_ref[...] = jnp.dot(fbn, w1_ref[...])

    mc = (mark2_ref[...] == 1).astype(jnp.float32)
    ii = lax.broadcasted_iota(jnp.int32, (128, 128), 0)
    jj = lax.broadcasted_iota(jnp.int32, (128, 128), 1)
    u_tri = (ii <= jj).astype(jnp.float32)
    rowcum = jnp.dot(mc, u_tri)
    s = rowcum[:, 127:128]
    ri = lax.broadcasted_iota(jnp.int32, (80, 80), 0)
    rj = lax.broadcasted_iota(jnp.int32, (80, 80), 1)
    l_tri = (rj < ri).astype(jnp.float32)
    p = jnp.dot(l_tri, s)
    cc = rowcum + p
    ncon = p[79:80, 0:1] + s[79:80, 0:1]
    gi = (lax.broadcasted_iota(jnp.int32, (80, 128), 0) * 128
          + lax.broadcasted_iota(jnp.int32, (80, 128), 1)).astype(jnp.float32)
    inv = jnp.where(mc == 1.0, cc - 1.0, ncon + gi - cc)
    inv_ref[...] = inv.astype(jnp.int32)


def _k1(x_pad, mark_pad, W_var, b_var, W_con, b_con, g1, be1, W1):
    return pl.pallas_call(
        _k1_body,
        out_shape=(jax.ShapeDtypeStruct((NP, H), jnp.float32),
                   jax.ShapeDtypeStruct((80, 128), jnp.int32)),
        interpret=_INTERPRET,
    )(x_pad, mark_pad.reshape(NP, 1), mark_pad.reshape(80, 128),
      W_var, b_var.reshape(1, H), W_con, b_con.reshape(1, H),
      g1.reshape(1, H), be1.reshape(1, H), W1)


def _sc2_body(rowr, colr, ewr, zsr, zeros2r, accout,
              accsp, rowbuf, colbuf, ewbuf, rowsbuf, sem):
    c = lax.axis_index("c")
    s = lax.axis_index("s")
    wid = c * 16 + s

    pltpu.sync_copy(zeros2r, accsp.at[pl.ds(s * 640, 640)])
    plsc.subcore_barrier()

    ebase = wid * EPW

    @pl.loop(0, NW_EDGE)
    def _(j):
        off = ebase + j * 128
        pltpu.sync_copy(rowr.at[pl.ds(off, 128)], rowbuf)
        pltpu.sync_copy(colr.at[pl.ds(off, 128)], colbuf)
        pltpu.sync_copy(ewr.at[pl.ds(off, 128)], ewbuf)
        pltpu.async_copy(zsr.at[rowbuf], rowsbuf, sem).wait()
        for t in range(8):
            wv = ewbuf[pl.ds(16 * t, 16)]
            for e in range(16):
                sc = wv[e]
                r = 16 * t + e
                for k in range(8):
                    rowsbuf[r, pl.ds(16 * k, 16)] = (
                        rowsbuf[r, pl.ds(16 * k, 16)] * sc)
        pltpu.sync_copy(rowsbuf, accsp.at[colbuf], add=True)

    plsc.subcore_barrier()
    pltpu.sync_copy(accsp.at[pl.ds(s * 640, 640)],
                    accout.at[c, pl.ds(s * 640, 640)])


def _sc2(row_p, col_p, ew_p, zs_p, zeros2):
    return pl.kernel(
        _sc2_body,
        out_type=jax.ShapeDtypeStruct((2, NP, H), jnp.float32),
        mesh=_SC_MESH,
        scratch_types=[
            pltpu.VMEM_SHARED((NP, H), jnp.float32),
            pltpu.VMEM((128,), jnp.int32),
            pltpu.VMEM((128,), jnp.int32),
            pltpu.VMEM((128,), jnp.float32),
            pltpu.VMEM((128, H), jnp.float32),
            pltpu.SemaphoreType.DMA,
        ],
    )(row_p, col_p, ew_p, zs_p, zeros2)


def _k3_body(acc_ref, dis_ref, b1_ref, g2_ref, be2_ref, h2_ref):
    y = dis_ref[...] * acc_ref[...] + b1_ref[...]
    rows = lax.broadcasted_iota(jnp.int32, (NP, 1), 0)
    real = rows < N
    ym = jnp.where(real, y, 0.0)
    m = jnp.sum(ym, axis=0, keepdims=True) / N
    v = jnp.sum(jnp.where(real, (y - m) ** 2, 0.0), axis=0, keepdims=True) / N
    h2_ref[...] = jnp.tanh((y - m) * lax.rsqrt(v + 1e-5) * g2_ref[...]
                           + be2_ref[...])


def _k3(acc, dis, b1, g2, be2):
    return pl.pallas_call(
        _k3_body,
        out_shape=jax.ShapeDtypeStruct((NP, H), jnp.float32),
        interpret=_INTERPRET,
    )(acc, dis.reshape(NP, 1), b1.reshape(1, H), g2.reshape(1, H),
      be2.reshape(1, H))


def _k4_body(psum_ref, cnt_ref, adj_ref, w2_ref, b2_ref, w3_ref, b3_ref,
             g3_ref, be3_ref, feat_ref, mean_ref):
    pooled = psum_ref[...] / jnp.maximum(cnt_ref[...], 1.0)
    a = (adj_ref[...] >= 0.7).astype(jnp.float32)
    ri = lax.broadcasted_iota(jnp.int32, (S, S), 0)
    ci = lax.broadcasted_iota(jnp.int32, (S, S), 1)
    ah = a + (ri == ci).astype(jnp.float32)
    deg_row = jnp.sum(ah, axis=0, keepdims=True)
    ones_col = jnp.ones((S, 1), jnp.float32)
    deg_col = lax.dot_general(ah, ones_col,
                              (((0,), (0,)), ((), ())))
    m = lax.rsqrt(deg_col) * ah * lax.rsqrt(deg_row)
    xw2 = jnp.dot(pooled, w2_ref[...])
    out2 = lax.dot_general(m, xw2, (((0,), (0,)), ((), ())), precision=lax.Precision.HIGHEST) + b2_ref[...]
    mm = jnp.mean(out2, axis=0, keepdims=True)
    vv = jnp.mean((out2 - mm) ** 2, axis=0, keepdims=True)
    f1 = jnp.tanh((out2 - mm) * lax.rsqrt(vv + 1e-5) * g3_ref[...]
                  + be3_ref[...])
    xw3 = jnp.dot(f1, w3_ref[...])
    out3 = jnp.tanh(lax.dot_general(m, xw3, (((0,), (0,)), ((), ())), precision=lax.Precision.HIGHEST)
                    + b3_ref[...])
    feat_ref[...] = out3
    mean_ref[...] = jnp.mean(out3, axis=0, keepdims=True)


def _k4(pool_sum, cnt, scen_adj, W2, b2, W3, b3, g3, be3):
    return pl.pallas_call(
        _k4_body,
        out_shape=(jax.ShapeDtypeStruct((S, H), jnp.float32),
                   jax.ShapeDtypeStruct((1, H), jnp.float32)),
        interpret=_INTERPRET,
    )(pool_sum, cnt, scen_adj, W2, b2.reshape(1, H), W3, b3.reshape(1, H),
      g3.reshape(1, H), be3.reshape(1, H))


def kernel(x, mark, edge_index, edge_attr, batch, scen_adj,
           W_var, b_var, W_con, b_con, W1, b1, W2, b2, W3, b3,
           g1, be1, g2, be2, g3, be3):
    f32, i32 = jnp.float32, jnp.int32
    mark = mark.astype(i32)

    x_pad = jnp.concatenate([x, jnp.zeros((NP - N, D), f32)])
    mark_pad = jnp.concatenate([mark, jnp.full((NP - N,), 2, i32)])

    npad_e = EP - 320000 - N
    dummy = (N + (jnp.arange(npad_e, dtype=i32) % (NP - N)))
    row_p = jnp.concatenate([edge_index[0].astype(i32),
                             jnp.arange(N, dtype=i32), dummy])
    col_p = jnp.concatenate([edge_index[1].astype(i32),
                             jnp.arange(N, dtype=i32), dummy])
    ew_p = jnp.concatenate([edge_attr.astype(f32), jnp.ones((N,), f32),
                            jnp.zeros((npad_e,), f32)])
    batch_pad = jnp.concatenate(
        [batch.astype(i32), S + (jnp.arange(NP - N, dtype=i32) % (SP - S))])

    z, inv2 = _k1(x_pad, mark_pad, W_var, b_var, W_con, b_con, g1, be1, W1)
    inv = inv2.reshape(NP)

    zeros_np = jnp.zeros((NP,), f32)
    deg_part, z_p = _sc1(col_p, ew_p, z, inv, zeros_np)
    deg = deg_part[0] + deg_part[1]
    dis = jnp.where(deg > 0, lax.rsqrt(jnp.maximum(deg, 1e-30)), 0.0)
    zs_p = dis[:, None] * z_p
    zeros2 = jnp.zeros((640, H), f32)
    acc_part = _sc2(row_p, col_p, ew_p, zs_p, zeros2)
    acc = acc_part[0] + acc_part[1]

    h2 = _k3(acc, dis, b1, g2, be2)

    pool_sum = jax.ops.segment_sum(h2, batch_pad, num_segments=SP)[:S]
    cnt = jax.ops.segment_sum(jnp.ones((NP,), f32), batch_pad,
                              num_segments=SP)[:S]

    feat, mean = _k4(pool_sum, cnt.reshape(S, 1), scen_adj, W2, b2, W3, b3,
                     g3, be3)
    return (feat, mean.reshape(H))


# speedup vs baseline: 24.2059x; 1.1398x over previous
import functools

import jax
import jax.numpy as jnp
from jax import lax
from jax.experimental import pallas as pl
from jax.experimental.pallas import tpu as pltpu
from jax.experimental.pallas import tpu_sc as plsc

_INTERPRET = False

N = 10000
NP = 10240
D = 128
H = 128
S = 512
SP = 640
EP = 331776
EPW = EP // 32
NW_EDGE = EPW // 128
ROWS_W = NP // 32

_SC_MESH = plsc.VectorSubcoreMesh(core_axis_name="c", subcore_axis_name="s")


def _sc1_body(colr, ewr, zr, invr, zerosr, degout, zpout,
              degacc, colbuf, ewbuf, zbuf, invbuf, sem):
    c = lax.axis_index("c")
    s = lax.axis_index("s")
    wid = c * 16 + s

    @pl.when(s == 0)
    def _():
        pltpu.sync_copy(zerosr, degacc)

    plsc.subcore_barrier()
    ebase = wid * EPW

    @pl.loop(0, NW_EDGE)
    def _(j):
        off = ebase + j * 128
        pltpu.sync_copy(colr.at[pl.ds(off, 128)], colbuf)
        pltpu.sync_copy(ewr.at[pl.ds(off, 128)], ewbuf)
        pltpu.sync_copy(ewbuf, degacc.at[colbuf], add=True)

    rbase = wid * ROWS_W
    for q in range(4):
        pltpu.sync_copy(invr.at[pl.ds(rbase + 80 * q, 80)], invbuf)
        pltpu.sync_copy(zr.at[pl.ds(rbase + 80 * q, 80)], zbuf)
        pltpu.async_copy(zbuf, zpout.at[invbuf], sem).wait()

    plsc.subcore_barrier()

    @pl.when(s == 0)
    def _():
        pltpu.sync_copy(degacc, degout.at[c])


def _sc1(col_p, ew_p, z, inv, zeros_np):
    return pl.kernel(
        _sc1_body,
        out_type=(jax.ShapeDtypeStruct((2, NP), jnp.float32),
                  jax.ShapeDtypeStruct((NP, H), jnp.float32)),
        mesh=_SC_MESH,
        scratch_types=[
            pltpu.VMEM_SHARED((NP,), jnp.float32),
            pltpu.VMEM((128,), jnp.int32),
            pltpu.VMEM((128,), jnp.float32),
            pltpu.VMEM((80, H), jnp.float32),
            pltpu.VMEM((80,), jnp.int32),
            pltpu.SemaphoreType.DMA,
        ],
    )(col_p, ew_p, z, inv, zeros_np)


def _k1_body(x_ref, markc_ref, mark2_ref, wv_ref, bv_ref, wc_ref, bc_ref,
             g1_ref, be1_ref, w1_ref, z_ref, inv_ref):
    x = x_ref[...]
    markc = markc_ref[...]
    var_all = jnp.tanh(jnp.dot(x, wv_ref[...]) + bv_ref[...])
    con_all = jnp.tanh(x[:, 0:1] * wc_ref[...] + bc_ref[...])
    feats = jnp.where(markc == 1, con_all, var_all)
    rows = lax.broadcasted_iota(jnp.int32, (NP, 1), 0)
    real = rows < N
    fm = jnp.where(real, feats, 0.0)
    m = jnp.sum(fm, axis=0, keepdims=True) / N
    v = jnp.sum(jnp.where(real, (feats - m) ** 2, 0.0), axis=0,
                keepdims=True) / N
    fbn = (feats - m) * lax.rsqrt(v + 1e-5) * g1_ref[...] + be1_ref[...]
    z_ref[...] = jnp.dot(fbn, w1_ref[...])

    mc = (mark2_ref[...] == 1).astype(jnp.float32)
    ii = lax.broadcasted_iota(jnp.int32, (128, 128), 0)
    jj = lax.broadcasted_iota(jnp.int32, (128, 128), 1)
    u_tri = (ii <= jj).astype(jnp.float32)
    rowcum = jnp.dot(mc, u_tri)
    s = rowcum[:, 127:128]
    ri = lax.broadcasted_iota(jnp.int32, (80, 80), 0)
    rj = lax.broadcasted_iota(jnp.int32, (80, 80), 1)
    l_tri = (rj < ri).astype(jnp.float32)
    p = jnp.dot(l_tri, s)
    cc = rowcum + p
    ncon = p[79:80, 0:1] + s[79:80, 0:1]
    gi = (lax.broadcasted_iota(jnp.int32, (80, 128), 0) * 128
          + lax.broadcasted_iota(jnp.int32, (80, 128), 1)).astype(jnp.float32)
    inv = jnp.where(mc == 1.0, cc - 1.0, ncon + gi - cc)
    inv_ref[...] = inv.astype(jnp.int32)


def _k1(x_pad, mark_pad, W_var, b_var, W_con, b_con, g1, be1, W1):
    return pl.pallas_call(
        _k1_body,
        out_shape=(jax.ShapeDtypeStruct((NP, H), jnp.float32),
                   jax.ShapeDtypeStruct((80, 128), jnp.int32)),
        interpret=_INTERPRET,
    )(x_pad, mark_pad.reshape(NP, 1), mark_pad.reshape(80, 128),
      W_var, b_var.reshape(1, H), W_con, b_con.reshape(1, H),
      g1.reshape(1, H), be1.reshape(1, H), W1)


def _sc2_body(rowr, colr, ewr, zsr, zeros2r, accout,
              accsp, rowbuf, colbuf, ewbuf, rowsbuf, sem):
    c = lax.axis_index("c")
    s = lax.axis_index("s")
    wid = c * 16 + s

    pltpu.sync_copy(zeros2r, accsp.at[pl.ds(s * 640, 640)])
    plsc.subcore_barrier()

    ebase = wid * EPW

    @pl.loop(0, NW_EDGE)
    def _(j):
        off = ebase + j * 128
        pltpu.sync_copy(rowr.at[pl.ds(off, 128)], rowbuf)
        pltpu.sync_copy(colr.at[pl.ds(off, 128)], colbuf)
        pltpu.sync_copy(ewr.at[pl.ds(off, 128)], ewbuf)
        pltpu.async_copy(zsr.at[rowbuf], rowsbuf, sem).wait()
        for t in range(8):
            wv = ewbuf[pl.ds(16 * t, 16)]
            for e in range(16):
                sc = wv[e]
                r = 16 * t + e
                for k in range(8):
                    rowsbuf[r, pl.ds(16 * k, 16)] = (
                        rowsbuf[r, pl.ds(16 * k, 16)] * sc)
        pltpu.sync_copy(rowsbuf, accsp.at[colbuf], add=True)

    plsc.subcore_barrier()
    pltpu.sync_copy(accsp.at[pl.ds(s * 640, 640)],
                    accout.at[c, pl.ds(s * 640, 640)])


def _sc2(row_p, col_p, ew_p, zs_p, zeros2):
    return pl.kernel(
        _sc2_body,
        out_type=jax.ShapeDtypeStruct((2, NP, H), jnp.float32),
        mesh=_SC_MESH,
        scratch_types=[
            pltpu.VMEM_SHARED((NP, H), jnp.float32),
            pltpu.VMEM((128,), jnp.int32),
            pltpu.VMEM((128,), jnp.int32),
            pltpu.VMEM((128,), jnp.float32),
            pltpu.VMEM((128, H), jnp.float32),
            pltpu.SemaphoreType.DMA,
        ],
    )(row_p, col_p, ew_p, zs_p, zeros2)


def _sc3_body(h2r, batchr, onesr, zeros2r, zcr, poolout, cntout,
              poolacc, cntacc, hbuf, bbuf, onesbuf, sem):
    c = lax.axis_index("c")
    s = lax.axis_index("s")
    wid = c * 16 + s

    pltpu.sync_copy(zeros2r.at[pl.ds(40 * s, 40)],
                    poolacc.at[pl.ds(40 * s, 40)])
    pltpu.sync_copy(onesr, onesbuf)

    @pl.when(s == 0)
    def _():
        pltpu.sync_copy(zcr, cntacc)

    plsc.subcore_barrier()
    rbase = wid * ROWS_W
    for q in range(4):
        pltpu.sync_copy(batchr.at[pl.ds(rbase + 80 * q, 80)], bbuf)
        pltpu.sync_copy(h2r.at[pl.ds(rbase + 80 * q, 80)], hbuf)
        pltpu.sync_copy(hbuf, poolacc.at[bbuf], add=True)
        pltpu.sync_copy(onesbuf, cntacc.at[bbuf], add=True)
    plsc.subcore_barrier()
    pltpu.sync_copy(poolacc.at[pl.ds(40 * s, 40)],
                    poolout.at[c, pl.ds(40 * s, 40)])

    @pl.when(s == 0)
    def _():
        pltpu.sync_copy(cntacc, cntout.at[c])


def _sc3(h2, batch_pad, ones80, zeros2, zc):
    return pl.kernel(
        _sc3_body,
        out_type=(jax.ShapeDtypeStruct((2, SP, H), jnp.float32),
                  jax.ShapeDtypeStruct((2, SP), jnp.float32)),
        mesh=_SC_MESH,
        scratch_types=[
            pltpu.VMEM_SHARED((SP, H), jnp.float32),
            pltpu.VMEM_SHARED((SP,), jnp.float32),
            pltpu.VMEM((80, H), jnp.float32),
            pltpu.VMEM((80,), jnp.int32),
            pltpu.VMEM((80,), jnp.float32),
            pltpu.SemaphoreType.DMA,
        ],
    )(h2, batch_pad, ones80, zeros2, zc)


def _k3_body(acc_ref, dis_ref, b1_ref, g2_ref, be2_ref, h2_ref):
    y = dis_ref[...] * acc_ref[...] + b1_ref[...]
    rows = lax.broadcasted_iota(jnp.int32, (NP, 1), 0)
    real = rows < N
    ym = jnp.where(real, y, 0.0)
    m = jnp.sum(ym, axis=0, keepdims=True) / N
    v = jnp.sum(jnp.where(real, (y - m) ** 2, 0.0), axis=0, keepdims=True) / N
    h2_ref[...] = jnp.tanh((y - m) * lax.rsqrt(v + 1e-5) * g2_ref[...]
                           + be2_ref[...])


def _k3(acc, dis, b1, g2, be2):
    return pl.pallas_call(
        _k3_body,
        out_shape=jax.ShapeDtypeStruct((NP, H), jnp.float32),
        interpret=_INTERPRET,
    )(acc, dis.reshape(NP, 1), b1.reshape(1, H), g2.reshape(1, H),
      be2.reshape(1, H))


def _k4_body(psum_ref, cnt_ref, adj_ref, w2_ref, b2_ref, w3_ref, b3_ref,
             g3_ref, be3_ref, feat_ref, mean_ref):
    pooled = psum_ref[...] / jnp.maximum(cnt_ref[...], 1.0)
    a = (adj_ref[...] >= 0.7).astype(jnp.float32)
    ri = lax.broadcasted_iota(jnp.int32, (S, S), 0)
    ci = lax.broadcasted_iota(jnp.int32, (S, S), 1)
    ah = a + (ri == ci).astype(jnp.float32)
    deg_row = jnp.sum(ah, axis=0, keepdims=True)
    ones_col = jnp.ones((S, 1), jnp.float32)
    deg_col = lax.dot_general(ah, ones_col,
                              (((0,), (0,)), ((), ())))
    m = lax.rsqrt(deg_col) * ah * lax.rsqrt(deg_row)
    xw2 = jnp.dot(pooled, w2_ref[...])
    out2 = lax.dot_general(m, xw2, (((0,), (0,)), ((), ())), precision=lax.Precision.HIGHEST) + b2_ref[...]
    mm = jnp.mean(out2, axis=0, keepdims=True)
    vv = jnp.mean((out2 - mm) ** 2, axis=0, keepdims=True)
    f1 = jnp.tanh((out2 - mm) * lax.rsqrt(vv + 1e-5) * g3_ref[...]
                  + be3_ref[...])
    xw3 = jnp.dot(f1, w3_ref[...])
    out3 = jnp.tanh(lax.dot_general(m, xw3, (((0,), (0,)), ((), ())), precision=lax.Precision.HIGHEST)
                    + b3_ref[...])
    feat_ref[...] = out3
    mean_ref[...] = jnp.mean(out3, axis=0, keepdims=True)


def _k4(pool_sum, cnt, scen_adj, W2, b2, W3, b3, g3, be3):
    return pl.pallas_call(
        _k4_body,
        out_shape=(jax.ShapeDtypeStruct((S, H), jnp.float32),
                   jax.ShapeDtypeStruct((1, H), jnp.float32)),
        interpret=_INTERPRET,
    )(pool_sum, cnt, scen_adj, W2, b2.reshape(1, H), W3, b3.reshape(1, H),
      g3.reshape(1, H), be3.reshape(1, H))


def kernel(x, mark, edge_index, edge_attr, batch, scen_adj,
           W_var, b_var, W_con, b_con, W1, b1, W2, b2, W3, b3,
           g1, be1, g2, be2, g3, be3):
    f32, i32 = jnp.float32, jnp.int32
    mark = mark.astype(i32)

    x_pad = jnp.concatenate([x, jnp.zeros((NP - N, D), f32)])
    mark_pad = jnp.concatenate([mark, jnp.full((NP - N,), 2, i32)])

    npad_e = EP - 320000 - N
    dummy = (N + (jnp.arange(npad_e, dtype=i32) % (NP - N)))
    row_p = jnp.concatenate([edge_index[0].astype(i32),
                             jnp.arange(N, dtype=i32), dummy])
    col_p = jnp.concatenate([edge_index[1].astype(i32),
                             jnp.arange(N, dtype=i32), dummy])
    ew_p = jnp.concatenate([edge_attr.astype(f32), jnp.ones((N,), f32),
                            jnp.zeros((npad_e,), f32)])
    batch_pad = jnp.concatenate(
        [batch.astype(i32), S + (jnp.arange(NP - N, dtype=i32) % (SP - S))])

    z, inv2 = _k1(x_pad, mark_pad, W_var, b_var, W_con, b_con, g1, be1, W1)
    inv = inv2.reshape(NP)

    zeros_np = jnp.zeros((NP,), f32)
    deg_part, z_p = _sc1(col_p, ew_p, z, inv, zeros_np)
    deg = deg_part[0] + deg_part[1]
    dis = jnp.where(deg > 0, lax.rsqrt(jnp.maximum(deg, 1e-30)), 0.0)
    zs_p = dis[:, None] * z_p
    zeros2 = jnp.zeros((640, H), f32)
    acc_part = _sc2(row_p, col_p, ew_p, zs_p, zeros2)
    acc = acc_part[0] + acc_part[1]

    h2 = _k3(acc, dis, b1, g2, be2)

    pool_part, cnt_part = _sc3(h2, batch_pad, jnp.ones((80,), f32), zeros2,
                               jnp.zeros((SP,), f32))
    pool_sum = pool_part[0, :S] + pool_part[1, :S]
    cnt = cnt_part[0, :S] + cnt_part[1, :S]

    feat, mean = _k4(pool_sum, cnt.reshape(S, 1), scen_adj, W2, b2, W3, b3,
                     g3, be3)
    return (feat, mean.reshape(H))


# speedup vs baseline: 42.1056x; 1.7395x over previous
import functools

import jax
import jax.numpy as jnp
from jax import lax
from jax.experimental import pallas as pl
from jax.experimental.pallas import tpu as pltpu
from jax.experimental.pallas import tpu_sc as plsc

_INTERPRET = False

N = 10000
NP = 10240
D = 128
H = 128
S = 512
SP = 640
EP = 344064
EPW = EP // 32
NW_EDGE = EPW // 128
ROWS_W = NP // 32

_SC_MESH = plsc.VectorSubcoreMesh(core_axis_name="c", subcore_axis_name="s")


def _sc1_body(colr, ewr, zr, invr, zerosr, degout, zpout,
              degacc, colb0, colb1, ewb0, ewb1, zbuf, invbuf,
              semi, sems, semz):
    c = lax.axis_index("c")
    s = lax.axis_index("s")
    wid = c * 16 + s
    colb = (colb0, colb1)
    ewb = (ewb0, ewb1)

    @pl.when(s == 0)
    def _():
        pltpu.sync_copy(zerosr, degacc)

    plsc.subcore_barrier()
    ebase = wid * EPW

    def start_idx(j, b):
        off = ebase + j * 128
        pltpu.async_copy(colr.at[pl.ds(off, 128)], colb[b], semi)
        pltpu.async_copy(ewr.at[pl.ds(off, 128)], ewb[b], semi)

    def wait_idx(b):
        pltpu.make_async_copy(colr.at[pl.ds(0, 128)], colb[b], semi).wait()
        pltpu.make_async_copy(ewr.at[pl.ds(0, 128)], ewb[b], semi).wait()

    def wait_scat(b):
        pltpu.make_async_copy(ewb[b], degacc.at[colb[b]], sems).wait()

    start_idx(0, 0)

    @pl.loop(0, NW_EDGE, step=2)
    def _(j0):
        for b in range(2):
            j = j0 + b
            wait_idx(b)

            @pl.when(j > 0)
            def _():
                wait_scat(1 - b)

            @pl.when(j < NW_EDGE - 1)
            def _():
                start_idx(j + 1, 1 - b)

            pltpu.async_copy(ewb[b], degacc.at[colb[b]], sems, add=True)

    wait_scat((NW_EDGE - 1) % 2)

    rbase = wid * ROWS_W
    for q in range(4):
        pltpu.sync_copy(invr.at[pl.ds(rbase + 80 * q, 80)], invbuf)
        pltpu.sync_copy(zr.at[pl.ds(rbase + 80 * q, 80)], zbuf)
        pltpu.async_copy(zbuf, zpout.at[invbuf], semz).wait()

    plsc.subcore_barrier()

    @pl.when(s == 0)
    def _():
        pltpu.sync_copy(degacc, degout.at[c])


def _sc1(col_p, ew_p, z, inv, zeros_np):
    return pl.kernel(
        _sc1_body,
        out_type=(jax.ShapeDtypeStruct((2, NP), jnp.float32),
                  jax.ShapeDtypeStruct((NP, H), jnp.float32)),
        mesh=_SC_MESH,
        scratch_types=[
            pltpu.VMEM_SHARED((NP,), jnp.float32),
            pltpu.VMEM((128,), jnp.int32),
            pltpu.VMEM((128,), jnp.int32),
            pltpu.VMEM((128,), jnp.float32),
            pltpu.VMEM((128,), jnp.float32),
            pltpu.VMEM((80, H), jnp.float32),
            pltpu.VMEM((80,), jnp.int32),
            pltpu.SemaphoreType.DMA,
            pltpu.SemaphoreType.DMA,
            pltpu.SemaphoreType.DMA,
        ],
    )(col_p, ew_p, z, inv, zeros_np)


def _k1_body(x_ref, markc_ref, mark2_ref, wv_ref, bv_ref, wc_ref, bc_ref,
             g1_ref, be1_ref, w1_ref, z_ref, inv_ref):
    x = x_ref[...]
    markc = markc_ref[...]
    var_all = jnp.tanh(jnp.dot(x, wv_ref[...]) + bv_ref[...])
    con_all = jnp.tanh(x[:, 0:1] * wc_ref[...] + bc_ref[...])
    feats = jnp.where(markc == 1, con_all, var_all)
    rows = lax.broadcasted_iota(jnp.int32, (NP, 1), 0)
    real = rows < N
    fm = jnp.where(real, feats, 0.0)
    m = jnp.sum(fm, axis=0, keepdims=True) / N
    v = jnp.sum(jnp.where(real, (feats - m) ** 2, 0.0), axis=0,
                keepdims=True) / N
    fbn = (feats - m) * lax.rsqrt(v + 1e-5) * g1_ref[...] + be1_ref[...]
    z_ref[...] = jnp.dot(fbn, w1_ref[...])

    mc = (mark2_ref[...] == 1).astype(jnp.float32)
    ii = lax.broadcasted_iota(jnp.int32, (128, 128), 0)
    jj = lax.broadcasted_iota(jnp.int32, (128, 128), 1)
    u_tri = (ii <= jj).astype(jnp.float32)
    rowcum = jnp.dot(mc, u_tri)
    s = rowcum[:, 127:128]
    ri = lax.broadcasted_iota(jnp.int32, (80, 80), 0)
    rj = lax.broadcasted_iota(jnp.int32, (80, 80), 1)
    l_tri = (rj < ri).astype(jnp.float32)
    p = jnp.dot(l_tri, s)
    cc = rowcum + p
    ncon = p[79:80, 0:1] + s[79:80, 0:1]
    gi = (lax.broadcasted_iota(jnp.int32, (80, 128), 0) * 128
          + lax.broadcasted_iota(jnp.int32, (80, 128), 1)).astype(jnp.float32)
    inv = jnp.where(mc == 1.0, cc - 1.0, ncon + gi - cc)
    inv_ref[...] = inv.astype(jnp.int32)


def _k1(x_pad, mark_pad, W_var, b_var, W_con, b_con, g1, be1, W1):
    return pl.pallas_call(
        _k1_body,
        out_shape=(jax.ShapeDtypeStruct((NP, H), jnp.float32),
                   jax.ShapeDtypeStruct((80, 128), jnp.int32)),
        interpret=_INTERPRET,
    )(x_pad, mark_pad.reshape(NP, 1), mark_pad.reshape(80, 128),
      W_var, b_var.reshape(1, H), W_con, b_con.reshape(1, H),
      g1.reshape(1, H), be1.reshape(1, H), W1)


def _sc2_body(rowr, colr, ewr2, zsr, zeros2r, accout,
              accsp, rowb0, rowb1, colb0, colb1, ewb0, ewb1,
              scolb0, scolb1, sewb0, sewb1, rs0, rs1, semi, semg, sems):
    c = lax.axis_index("c")
    s = lax.axis_index("s")
    wid = c * 16 + s
    rowb = (rowb0, rowb1)
    colb = (colb0, colb1)
    ewb = (ewb0, ewb1)
    scolb = (scolb0, scolb1)
    sewb = (sewb0, sewb1)
    rs = (rs0, rs1)

    pltpu.sync_copy(zeros2r, accsp.at[pl.ds(s * 640, 640)])
    plsc.subcore_barrier()

    ebase = wid * EPW

    def start_idx(j, b):
        off = pl.multiple_of(ebase + j * 128, 128)
        off8 = pl.multiple_of((ebase + j * 128) // 16, 8)
        pltpu.async_copy(rowr.at[pl.ds(off, 128)], rowb[b], semi)
        pltpu.async_copy(colr.at[pl.ds(off, 128)], colb[b], semi)
        pltpu.async_copy(ewr2.at[pl.ds(off8, 8)], ewb[b], semi)

    def wait_idx(b):
        pltpu.make_async_copy(rowr.at[pl.ds(0, 128)], rowb[b], semi).wait()
        pltpu.make_async_copy(colr.at[pl.ds(0, 128)], colb[b], semi).wait()
        pltpu.make_async_copy(ewr2.at[pl.ds(0, 8)], ewb[b], semi).wait()

    def start_gather(b):
        pltpu.async_copy(zsr.at[rowb[b]], rs[b], semg)

    def wait_gather(b):
        pltpu.make_async_copy(zsr.at[rowb[b]], rs[b], semg).wait()

    def start_scatter(b):
        pltpu.async_copy(rs[b], accsp.at[scolb[b]], sems, add=True)

    def wait_scatter(b):
        pltpu.make_async_copy(rs[b], accsp.at[scolb[b]], sems).wait()

    start_idx(0, 0)
    wait_idx(0)
    start_gather(0)
    start_idx(1, 1)

    @pl.loop(0, NW_EDGE, step=2)
    def _(j0):
        for b in range(2):
            j = j0 + b
            oth = 1 - b
            wait_gather(b)

            @pl.when(j > 0)
            def _():
                wait_scatter(oth)

            @pl.when(j < NW_EDGE - 1)
            def _():
                wait_idx(oth)
                start_gather(oth)

            for i in range(8):
                scolb[b][pl.ds(16 * i, 16)] = colb[b][pl.ds(16 * i, 16)]
                sewb[b][i] = ewb[b][i]

            @pl.when(j < NW_EDGE - 2)
            def _():
                start_idx(j + 2, b)

            @pl.loop(0, 8)
            def _(t):
                wv = sewb[b][t]
                for e in range(16):
                    sc = wv[e]
                    r = 16 * t + e
                    for k in range(8):
                        rs[b][r, pl.ds(16 * k, 16)] = (
                            rs[b][r, pl.ds(16 * k, 16)] * sc)

            start_scatter(b)

    wait_scatter((NW_EDGE - 1) % 2)
    plsc.subcore_barrier()
    pltpu.sync_copy(accsp.at[pl.ds(s * 640, 640)],
                    accout.at[c, pl.ds(s * 640, 640)])


def _sc2(row_p, col_p, ew_p, zs_p, zeros2):
    return pl.kernel(
        _sc2_body,
        out_type=jax.ShapeDtypeStruct((2, NP, H), jnp.float32),
        mesh=_SC_MESH,
        scratch_types=[
            pltpu.VMEM_SHARED((NP, H), jnp.float32),
            pltpu.VMEM((128,), jnp.int32),
            pltpu.VMEM((128,), jnp.int32),
            pltpu.VMEM((128,), jnp.int32),
            pltpu.VMEM((128,), jnp.int32),
            pltpu.VMEM((8, 16), jnp.float32),
            pltpu.VMEM((8, 16), jnp.float32),
            pltpu.VMEM((128,), jnp.int32),
            pltpu.VMEM((128,), jnp.int32),
            pltpu.VMEM((8, 16), jnp.float32),
            pltpu.VMEM((8, 16), jnp.float32),
            pltpu.VMEM((128, H), jnp.float32),
            pltpu.VMEM((128, H), jnp.float32),
            pltpu.SemaphoreType.DMA,
            pltpu.SemaphoreType.DMA,
            pltpu.SemaphoreType.DMA,
        ],
    )(row_p, col_p, ew_p.reshape(EP // 16, 16), zs_p, zeros2)


def _sc3_body(h2r, batchr, onesr, zeros2r, zcr, poolout, cntout,
              poolacc, cntacc, hbuf, bbuf, onesbuf, sem):
    c = lax.axis_index("c")
    s = lax.axis_index("s")
    wid = c * 16 + s

    pltpu.sync_copy(zeros2r.at[pl.ds(40 * s, 40)],
                    poolacc.at[pl.ds(40 * s, 40)])
    pltpu.sync_copy(onesr, onesbuf)

    @pl.when(s == 0)
    def _():
        pltpu.sync_copy(zcr, cntacc)

    plsc.subcore_barrier()
    rbase = wid * ROWS_W
    for q in range(4):
        pltpu.sync_copy(batchr.at[pl.ds(rbase + 80 * q, 80)], bbuf)
        pltpu.sync_copy(h2r.at[pl.ds(rbase + 80 * q, 80)], hbuf)
        pltpu.sync_copy(hbuf, poolacc.at[bbuf], add=True)
        pltpu.sync_copy(onesbuf, cntacc.at[bbuf], add=True)
    plsc.subcore_barrier()
    pltpu.sync_copy(poolacc.at[pl.ds(40 * s, 40)],
                    poolout.at[c, pl.ds(40 * s, 40)])

    @pl.when(s == 0)
    def _():
        pltpu.sync_copy(cntacc, cntout.at[c])


def _sc3(h2, batch_pad, ones80, zeros2, zc):
    return pl.kernel(
        _sc3_body,
        out_type=(jax.ShapeDtypeStruct((2, SP, H), jnp.float32),
                  jax.ShapeDtypeStruct((2, SP), jnp.float32)),
        mesh=_SC_MESH,
        scratch_types=[
            pltpu.VMEM_SHARED((SP, H), jnp.float32),
            pltpu.VMEM_SHARED((SP,), jnp.float32),
            pltpu.VMEM((80, H), jnp.float32),
            pltpu.VMEM((80,), jnp.int32),
            pltpu.VMEM((80,), jnp.float32),
            pltpu.SemaphoreType.DMA,
        ],
    )(h2, batch_pad, ones80, zeros2, zc)


def _k2_body(zp_ref, degp_ref, zs_ref, dis_ref):
    deg = degp_ref[:, 0:1] + degp_ref[:, 1:2]
    dis = jnp.where(deg > 0, lax.rsqrt(jnp.maximum(deg, 1e-30)), 0.0)
    dis_ref[...] = dis
    zs_ref[...] = dis * zp_ref[...]


def _k2(z_p, deg_part_t):
    return pl.pallas_call(
        _k2_body,
        out_shape=(jax.ShapeDtypeStruct((NP, H), jnp.float32),
                   jax.ShapeDtypeStruct((NP, 1), jnp.float32)),
        interpret=_INTERPRET,
    )(z_p, deg_part_t)


def _k3_body(accp_ref, dis_ref, b1_ref, g2_ref, be2_ref, h2_ref):
    acc = accp_ref[0] + accp_ref[1]
    y = dis_ref[...] * acc + b1_ref[...]
    rows = lax.broadcasted_iota(jnp.int32, (NP, 1), 0)
    real = rows < N
    ym = jnp.where(real, y, 0.0)
    m = jnp.sum(ym, axis=0, keepdims=True) / N
    v = jnp.sum(jnp.where(real, (y - m) ** 2, 0.0), axis=0, keepdims=True) / N
    h2_ref[...] = jnp.tanh((y - m) * lax.rsqrt(v + 1e-5) * g2_ref[...]
                           + be2_ref[...])


def _k3(acc_part, dis, b1, g2, be2):
    return pl.pallas_call(
        _k3_body,
        out_shape=jax.ShapeDtypeStruct((NP, H), jnp.float32),
        interpret=_INTERPRET,
    )(acc_part, dis, b1.reshape(1, H), g2.reshape(1, H),
      be2.reshape(1, H))


def _k4_body(psum_ref, cnt_ref, adj_ref, w2_ref, b2_ref, w3_ref, b3_ref,
             g3_ref, be3_ref, feat_ref, mean_ref):
    pooled = psum_ref[...] / jnp.maximum(cnt_ref[...], 1.0)
    a = (adj_ref[...] >= 0.7).astype(jnp.float32)
    ri = lax.broadcasted_iota(jnp.int32, (S, S), 0)
    ci = lax.broadcasted_iota(jnp.int32, (S, S), 1)
    ah = a + (ri == ci).astype(jnp.float32)
    deg_row = jnp.sum(ah, axis=0, keepdims=True)
    ones_col = jnp.ones((S, 1), jnp.float32)
    deg_col = lax.dot_general(ah, ones_col,
                              (((0,), (0,)), ((), ())))
    m = lax.rsqrt(deg_col) * ah * lax.rsqrt(deg_row)
    xw2 = jnp.dot(pooled, w2_ref[...])
    out2 = lax.dot_general(m, xw2, (((0,), (0,)), ((), ())), precision=lax.Precision.HIGHEST) + b2_ref[...]
    mm = jnp.mean(out2, axis=0, keepdims=True)
    vv = jnp.mean((out2 - mm) ** 2, axis=0, keepdims=True)
    f1 = jnp.tanh((out2 - mm) * lax.rsqrt(vv + 1e-5) * g3_ref[...]
                  + be3_ref[...])
    xw3 = jnp.dot(f1, w3_ref[...])
    out3 = jnp.tanh(lax.dot_general(m, xw3, (((0,), (0,)), ((), ())), precision=lax.Precision.HIGHEST)
                    + b3_ref[...])
    feat_ref[...] = out3
    mean_ref[...] = jnp.mean(out3, axis=0, keepdims=True)


def _k4(pool_sum, cnt, scen_adj, W2, b2, W3, b3, g3, be3):
    return pl.pallas_call(
        _k4_body,
        out_shape=(jax.ShapeDtypeStruct((S, H), jnp.float32),
                   jax.ShapeDtypeStruct((1, H), jnp.float32)),
        interpret=_INTERPRET,
    )(pool_sum, cnt, scen_adj, W2, b2.reshape(1, H), W3, b3.reshape(1, H),
      g3.reshape(1, H), be3.reshape(1, H))


def kernel(x, mark, edge_index, edge_attr, batch, scen_adj,
           W_var, b_var, W_con, b_con, W1, b1, W2, b2, W3, b3,
           g1, be1, g2, be2, g3, be3):
    f32, i32 = jnp.float32, jnp.int32
    mark = mark.astype(i32)

    x_pad = jnp.concatenate([x, jnp.zeros((NP - N, D), f32)])
    mark_pad = jnp.concatenate([mark, jnp.full((NP - N,), 2, i32)])

    npad_e = EP - 320000 - N
    dummy = (N + (jnp.arange(npad_e, dtype=i32) % (NP - N)))
    row_p = jnp.concatenate([edge_index[0].astype(i32),
                             jnp.arange(N, dtype=i32), dummy])
    col_p = jnp.concatenate([edge_index[1].astype(i32),
                             jnp.arange(N, dtype=i32), dummy])
    ew_p = jnp.concatenate([edge_attr.astype(f32), jnp.ones((N,), f32),
                            jnp.zeros((npad_e,), f32)])
    batch_pad = jnp.concatenate(
        [batch.astype(i32), S + (jnp.arange(NP - N, dtype=i32) % (SP - S))])

    z, inv2 = _k1(x_pad, mark_pad, W_var, b_var, W_con, b_con, g1, be1, W1)
    inv = inv2.reshape(NP)

    zeros_np = jnp.zeros((NP,), f32)
    deg_part, z_p = _sc1(col_p, ew_p, z, inv, zeros_np)

    zs_p, dis = _k2(z_p, deg_part.T)

    zeros2 = jnp.zeros((640, H), f32)
    acc_part = _sc2(row_p, col_p, ew_p, zs_p, zeros2)

    h2 = _k3(acc_part, dis, b1, g2, be2)

    pool_part, cnt_part = _sc3(h2, batch_pad, jnp.ones((80,), f32), zeros2,
                               jnp.zeros((SP,), f32))
    pool_sum = pool_part[0, :S] + pool_part[1, :S]
    cnt = cnt_part[0, :S] + cnt_part[1, :S]

    feat, mean = _k4(pool_sum, cnt.reshape(S, 1), scen_adj, W2, b2, W3, b3,
                     g3, be3)
    return (feat, mean.reshape(H))


# speedup vs baseline: 44.7014x; 1.0617x over previous
import functools

import jax
import jax.numpy as jnp
from jax import lax
from jax.experimental import pallas as pl
from jax.experimental.pallas import tpu as pltpu
from jax.experimental.pallas import tpu_sc as plsc

_INTERPRET = False

N = 10000
NP = 10240
D = 128
H = 128
S = 512
SP = 640
EP = 327680
EPW = EP // 32
NW_EDGE = EPW // 128
ROWS_W = NP // 32

_SC_MESH = plsc.VectorSubcoreMesh(core_axis_name="c", subcore_axis_name="s")


def _sc1_body(colr, ewr, zr, invr, zerosr, degout, zpout,
              degacc, colb0, colb1, ewb0, ewb1, zbuf, invbuf,
              semi, sems, semz):
    c = lax.axis_index("c")
    s = lax.axis_index("s")
    wid = c * 16 + s
    colb = (colb0, colb1)
    ewb = (ewb0, ewb1)

    @pl.when(s == 0)
    def _():
        pltpu.sync_copy(zerosr, degacc)

    plsc.subcore_barrier()
    ebase = wid * EPW

    def start_idx(j, b):
        off = ebase + j * 128
        pltpu.async_copy(colr.at[pl.ds(off, 128)], colb[b], semi)
        pltpu.async_copy(ewr.at[pl.ds(off, 128)], ewb[b], semi)

    def wait_idx(b):
        pltpu.make_async_copy(colr.at[pl.ds(0, 128)], colb[b], semi).wait()
        pltpu.make_async_copy(ewr.at[pl.ds(0, 128)], ewb[b], semi).wait()

    def wait_scat(b):
        pltpu.make_async_copy(ewb[b], degacc.at[colb[b]], sems).wait()

    start_idx(0, 0)

    @pl.loop(0, NW_EDGE, step=2)
    def _(j0):
        for b in range(2):
            j = j0 + b
            wait_idx(b)

            @pl.when(j > 0)
            def _():
                wait_scat(1 - b)

            @pl.when(j < NW_EDGE - 1)
            def _():
                start_idx(j + 1, 1 - b)

            pltpu.async_copy(ewb[b], degacc.at[colb[b]], sems, add=True)

    wait_scat((NW_EDGE - 1) % 2)

    rbase = wid * ROWS_W
    for q in range(4):
        pltpu.sync_copy(invr.at[pl.ds(rbase + 80 * q, 80)], invbuf)
        pltpu.sync_copy(zr.at[pl.ds(rbase + 80 * q, 80)], zbuf)
        pltpu.async_copy(zbuf, zpout.at[invbuf], semz).wait()

    plsc.subcore_barrier()

    @pl.when(s == 0)
    def _():
        pltpu.sync_copy(degacc, degout.at[c])


def _sc1(col_p, ew_p, z, inv, zeros_np):
    return pl.kernel(
        _sc1_body,
        out_type=(jax.ShapeDtypeStruct((2, NP), jnp.float32),
                  jax.ShapeDtypeStruct((NP, H), jnp.float32)),
        mesh=_SC_MESH,
        scratch_types=[
            pltpu.VMEM_SHARED((NP,), jnp.float32),
            pltpu.VMEM((128,), jnp.int32),
            pltpu.VMEM((128,), jnp.int32),
            pltpu.VMEM((128,), jnp.float32),
            pltpu.VMEM((128,), jnp.float32),
            pltpu.VMEM((80, H), jnp.float32),
            pltpu.VMEM((80,), jnp.int32),
            pltpu.SemaphoreType.DMA,
            pltpu.SemaphoreType.DMA,
            pltpu.SemaphoreType.DMA,
        ],
    )(col_p, ew_p, z, inv, zeros_np)


def _k1_body(x_ref, markc_ref, mark2_ref, wv_ref, bv_ref, wc_ref, bc_ref,
             g1_ref, be1_ref, w1_ref, z_ref, inv_ref):
    x = x_ref[...]
    markc = markc_ref[...]
    var_all = jnp.tanh(jnp.dot(x, wv_ref[...]) + bv_ref[...])
    con_all = jnp.tanh(x[:, 0:1] * wc_ref[...] + bc_ref[...])
    feats = jnp.where(markc == 1, con_all, var_all)
    rows = lax.broadcasted_iota(jnp.int32, (NP, 1), 0)
    real = rows < N
    fm = jnp.where(real, feats, 0.0)
    m = jnp.sum(fm, axis=0, keepdims=True) / N
    v = jnp.sum(jnp.where(real, (feats - m) ** 2, 0.0), axis=0,
                keepdims=True) / N
    fbn = (feats - m) * lax.rsqrt(v + 1e-5) * g1_ref[...] + be1_ref[...]
    z_ref[...] = jnp.dot(fbn, w1_ref[...])

    mc = (mark2_ref[...] == 1).astype(jnp.float32)
    ii = lax.broadcasted_iota(jnp.int32, (128, 128), 0)
    jj = lax.broadcasted_iota(jnp.int32, (128, 128), 1)
    u_tri = (ii <= jj).astype(jnp.float32)
    rowcum = jnp.dot(mc, u_tri)
    s = rowcum[:, 127:128]
    ri = lax.broadcasted_iota(jnp.int32, (80, 80), 0)
    rj = lax.broadcasted_iota(jnp.int32, (80, 80), 1)
    l_tri = (rj < ri).astype(jnp.float32)
    p = jnp.dot(l_tri, s)
    cc = rowcum + p
    ncon = p[79:80, 0:1] + s[79:80, 0:1]
    gi = (lax.broadcasted_iota(jnp.int32, (80, 128), 0) * 128
          + lax.broadcasted_iota(jnp.int32, (80, 128), 1)).astype(jnp.float32)
    inv = jnp.where(mc == 1.0, cc - 1.0, ncon + gi - cc)
    inv_ref[...] = inv.astype(jnp.int32)


def _k1(x_pad, mark_pad, W_var, b_var, W_con, b_con, g1, be1, W1):
    return pl.pallas_call(
        _k1_body,
        out_shape=(jax.ShapeDtypeStruct((NP, H), jnp.float32),
                   jax.ShapeDtypeStruct((80, 128), jnp.int32)),
        interpret=_INTERPRET,
    )(x_pad, mark_pad.reshape(NP, 1), mark_pad.reshape(80, 128),
      W_var, b_var.reshape(1, H), W_con, b_con.reshape(1, H),
      g1.reshape(1, H), be1.reshape(1, H), W1)


def _sc2_body(rowr, colr, ewr2, zsr, zeros2r, accout,
              accsp, rowb0, rowb1, colb0, colb1, ewb0, ewb1,
              scolb0, scolb1, sewb0, sewb1, rs0, rs1, semi, semg, sems):
    c = lax.axis_index("c")
    s = lax.axis_index("s")
    wid = c * 16 + s
    rowb = (rowb0, rowb1)
    colb = (colb0, colb1)
    ewb = (ewb0, ewb1)
    scolb = (scolb0, scolb1)
    sewb = (sewb0, sewb1)
    rs = (rs0, rs1)

    pltpu.sync_copy(zeros2r, accsp.at[pl.ds(s * 640, 640)])
    plsc.subcore_barrier()

    ebase = wid * EPW

    def start_idx(j, b):
        off = pl.multiple_of(ebase + j * 128, 128)
        off8 = pl.multiple_of((ebase + j * 128) // 16, 8)
        pltpu.async_copy(rowr.at[pl.ds(off, 128)], rowb[b], semi)
        pltpu.async_copy(colr.at[pl.ds(off, 128)], colb[b], semi)
        pltpu.async_copy(ewr2.at[pl.ds(off8, 8)], ewb[b], semi)

    def wait_idx(b):
        pltpu.make_async_copy(rowr.at[pl.ds(0, 128)], rowb[b], semi).wait()
        pltpu.make_async_copy(colr.at[pl.ds(0, 128)], colb[b], semi).wait()
        pltpu.make_async_copy(ewr2.at[pl.ds(0, 8)], ewb[b], semi).wait()

    def start_gather(b):
        pltpu.async_copy(zsr.at[rowb[b]], rs[b], semg)

    def wait_gather(b):
        pltpu.make_async_copy(zsr.at[rowb[b]], rs[b], semg).wait()

    def start_scatter(b):
        pltpu.async_copy(rs[b], accsp.at[scolb[b]], sems, add=True)

    def wait_scatter(b):
        pltpu.make_async_copy(rs[b], accsp.at[scolb[b]], sems).wait()

    start_idx(0, 0)
    wait_idx(0)
    start_gather(0)
    start_idx(1, 1)

    @pl.loop(0, NW_EDGE, step=2)
    def _(j0):
        for b in range(2):
            j = j0 + b
            oth = 1 - b
            wait_gather(b)

            @pl.when(j > 0)
            def _():
                wait_scatter(oth)

            @pl.when(j < NW_EDGE - 1)
            def _():
                wait_idx(oth)
                start_gather(oth)

            for i in range(8):
                scolb[b][pl.ds(16 * i, 16)] = colb[b][pl.ds(16 * i, 16)]
                sewb[b][i] = ewb[b][i]

            @pl.when(j < NW_EDGE - 2)
            def _():
                start_idx(j + 2, b)

            @pl.loop(0, 8)
            def _(t):
                wv = sewb[b][t]
                for e in range(16):
                    sc = wv[e]
                    r = 16 * t + e
                    for k in range(8):
                        rs[b][r, pl.ds(16 * k, 16)] = (
                            rs[b][r, pl.ds(16 * k, 16)] * sc)

            start_scatter(b)

    wait_scatter((NW_EDGE - 1) % 2)
    plsc.subcore_barrier()
    pltpu.sync_copy(accsp.at[pl.ds(s * 640, 640)],
                    accout.at[c, pl.ds(s * 640, 640)])


def _sc2(row_p, col_p, ew_p, zs_p, zeros2):
    return pl.kernel(
        _sc2_body,
        out_type=jax.ShapeDtypeStruct((2, NP, H), jnp.float32),
        mesh=_SC_MESH,
        scratch_types=[
            pltpu.VMEM_SHARED((NP, H), jnp.float32),
            pltpu.VMEM((128,), jnp.int32),
            pltpu.VMEM((128,), jnp.int32),
            pltpu.VMEM((128,), jnp.int32),
            pltpu.VMEM((128,), jnp.int32),
            pltpu.VMEM((8, 16), jnp.float32),
            pltpu.VMEM((8, 16), jnp.float32),
            pltpu.VMEM((128,), jnp.int32),
            pltpu.VMEM((128,), jnp.int32),
            pltpu.VMEM((8, 16), jnp.float32),
            pltpu.VMEM((8, 16), jnp.float32),
            pltpu.VMEM((128, H), jnp.float32),
            pltpu.VMEM((128, H), jnp.float32),
            pltpu.SemaphoreType.DMA,
            pltpu.SemaphoreType.DMA,
            pltpu.SemaphoreType.DMA,
        ],
    )(row_p, col_p, ew_p.reshape(EP // 16, 16), zs_p, zeros2)


def _sc3_body(h2r, batchr, onesr, zeros2r, zcr, poolout, cntout,
              poolacc, cntacc, hbuf, bbuf, onesbuf, sem):
    c = lax.axis_index("c")
    s = lax.axis_index("s")
    wid = c * 16 + s

    pltpu.sync_copy(zeros2r.at[pl.ds(40 * s, 40)],
                    poolacc.at[pl.ds(40 * s, 40)])
    pltpu.sync_copy(onesr, onesbuf)

    @pl.when(s == 0)
    def _():
        pltpu.sync_copy(zcr, cntacc)

    plsc.subcore_barrier()
    rbase = wid * ROWS_W
    for q in range(4):
        pltpu.sync_copy(batchr.at[pl.ds(rbase + 80 * q, 80)], bbuf)
        pltpu.sync_copy(h2r.at[pl.ds(rbase + 80 * q, 80)], hbuf)
        pltpu.sync_copy(hbuf, poolacc.at[bbuf], add=True)
        pltpu.sync_copy(onesbuf, cntacc.at[bbuf], add=True)
    plsc.subcore_barrier()
    pltpu.sync_copy(poolacc.at[pl.ds(40 * s, 40)],
                    poolout.at[c, pl.ds(40 * s, 40)])

    @pl.when(s == 0)
    def _():
        pltpu.sync_copy(cntacc, cntout.at[c])


def _sc3(h2, batch_pad, ones80, zeros2, zc):
    return pl.kernel(
        _sc3_body,
        out_type=(jax.ShapeDtypeStruct((2, SP, H), jnp.float32),
                  jax.ShapeDtypeStruct((2, SP), jnp.float32)),
        mesh=_SC_MESH,
        scratch_types=[
            pltpu.VMEM_SHARED((SP, H), jnp.float32),
            pltpu.VMEM_SHARED((SP,), jnp.float32),
            pltpu.VMEM((80, H), jnp.float32),
            pltpu.VMEM((80,), jnp.int32),
            pltpu.VMEM((80,), jnp.float32),
            pltpu.SemaphoreType.DMA,
        ],
    )(h2, batch_pad, ones80, zeros2, zc)


def _k2_body(zp_ref, degp_ref, zs_ref, dis_ref):
    deg = degp_ref[:, 0:1] + degp_ref[:, 1:2] + 1.0
    dis = lax.rsqrt(deg)
    dis_ref[...] = dis
    zs_ref[...] = dis * zp_ref[...]


def _k2(z_p, deg_part_t):
    return pl.pallas_call(
        _k2_body,
        out_shape=(jax.ShapeDtypeStruct((NP, H), jnp.float32),
                   jax.ShapeDtypeStruct((NP, 1), jnp.float32)),
        interpret=_INTERPRET,
    )(z_p, deg_part_t)


def _k3_body(accp_ref, zs_ref, dis_ref, b1_ref, g2_ref, be2_ref, h2_ref):
    acc = accp_ref[0] + accp_ref[1] + zs_ref[...]
    y = dis_ref[...] * acc + b1_ref[...]
    rows = lax.broadcasted_iota(jnp.int32, (NP, 1), 0)
    real = rows < N
    ym = jnp.where(real, y, 0.0)
    m = jnp.sum(ym, axis=0, keepdims=True) / N
    v = jnp.sum(jnp.where(real, (y - m) ** 2, 0.0), axis=0, keepdims=True) / N
    h2_ref[...] = jnp.tanh((y - m) * lax.rsqrt(v + 1e-5) * g2_ref[...]
                           + be2_ref[...])


def _k3(acc_part, zs, dis, b1, g2, be2):
    return pl.pallas_call(
        _k3_body,
        out_shape=jax.ShapeDtypeStruct((NP, H), jnp.float32),
        interpret=_INTERPRET,
    )(acc_part, zs, dis, b1.reshape(1, H), g2.reshape(1, H),
      be2.reshape(1, H))


def _k4_body(psum_ref, cnt_ref, adj_ref, w2_ref, b2_ref, w3_ref, b3_ref,
             g3_ref, be3_ref, feat_ref, mean_ref):
    pooled = psum_ref[...] / jnp.maximum(cnt_ref[...], 1.0)
    a = (adj_ref[...] >= 0.7).astype(jnp.float32)
    ri = lax.broadcasted_iota(jnp.int32, (S, S), 0)
    ci = lax.broadcasted_iota(jnp.int32, (S, S), 1)
    ah = a + (ri == ci).astype(jnp.float32)
    deg_row = jnp.sum(ah, axis=0, keepdims=True)
    ones_col = jnp.ones((S, 1), jnp.float32)
    deg_col = lax.dot_general(ah, ones_col,
                              (((0,), (0,)), ((), ())))
    m = lax.rsqrt(deg_col) * ah * lax.rsqrt(deg_row)
    xw2 = jnp.dot(pooled, w2_ref[...])
    out2 = lax.dot_general(m, xw2, (((0,), (0,)), ((), ())), precision=lax.Precision.HIGHEST) + b2_ref[...]
    mm = jnp.mean(out2, axis=0, keepdims=True)
    vv = jnp.mean((out2 - mm) ** 2, axis=0, keepdims=True)
    f1 = jnp.tanh((out2 - mm) * lax.rsqrt(vv + 1e-5) * g3_ref[...]
                  + be3_ref[...])
    xw3 = jnp.dot(f1, w3_ref[...])
    out3 = jnp.tanh(lax.dot_general(m, xw3, (((0,), (0,)), ((), ())), precision=lax.Precision.HIGHEST)
                    + b3_ref[...])
    feat_ref[...] = out3
    mean_ref[...] = jnp.mean(out3, axis=0, keepdims=True)


def _k4(pool_sum, cnt, scen_adj, W2, b2, W3, b3, g3, be3):
    return pl.pallas_call(
        _k4_body,
        out_shape=(jax.ShapeDtypeStruct((S, H), jnp.float32),
                   jax.ShapeDtypeStruct((1, H), jnp.float32)),
        interpret=_INTERPRET,
    )(pool_sum, cnt, scen_adj, W2, b2.reshape(1, H), W3, b3.reshape(1, H),
      g3.reshape(1, H), be3.reshape(1, H))


def kernel(x, mark, edge_index, edge_attr, batch, scen_adj,
           W_var, b_var, W_con, b_con, W1, b1, W2, b2, W3, b3,
           g1, be1, g2, be2, g3, be3):
    f32, i32 = jnp.float32, jnp.int32
    mark = mark.astype(i32)

    x_pad = jnp.concatenate([x, jnp.zeros((NP - N, D), f32)])
    mark_pad = jnp.concatenate([mark, jnp.full((NP - N,), 2, i32)])

    npad_e = EP - 320000
    dummy = (N + (jnp.arange(npad_e, dtype=i32) % (NP - N)))
    row_p = jnp.concatenate([edge_index[0].astype(i32), dummy])
    col_p = jnp.concatenate([edge_index[1].astype(i32), dummy])
    ew_p = jnp.concatenate([edge_attr.astype(f32),
                            jnp.zeros((npad_e,), f32)])
    batch_pad = jnp.concatenate(
        [batch.astype(i32), S + (jnp.arange(NP - N, dtype=i32) % (SP - S))])

    z, inv2 = _k1(x_pad, mark_pad, W_var, b_var, W_con, b_con, g1, be1, W1)
    inv = inv2.reshape(NP)

    zeros_np = jnp.zeros((NP,), f32)
    deg_part, z_p = _sc1(col_p, ew_p, z, inv, zeros_np)

    zs_p, dis = _k2(z_p, deg_part.T)

    zeros2 = jnp.zeros((640, H), f32)
    acc_part = _sc2(row_p, col_p, ew_p, zs_p, zeros2)

    h2 = _k3(acc_part, zs_p, dis, b1, g2, be2)

    pool_part, cnt_part = _sc3(h2, batch_pad, jnp.ones((80,), f32), zeros2,
                               jnp.zeros((SP,), f32))
    pool_sum = pool_part[0, :S] + pool_part[1, :S]
    cnt = cnt_part[0, :S] + cnt_part[1, :S]

    feat, mean = _k4(pool_sum, cnt.reshape(S, 1), scen_adj, W2, b2, W3, b3,
                     g3, be3)
    return (feat, mean.reshape(H))
